# Initial kernel scaffold; baseline (speedup 1.0000x reference)
import jax
import jax.numpy as jnp
from jax.experimental import pallas as pl


def kernel(node_embs, edge_embs, edge_index, W_rb, W_rb_inv, W_basis, W_self_w, W_self_b):
    raise NotImplementedError("write your pallas kernel here")


# speedup vs baseline: 2.1616x; 2.1616x over previous
import jax
import jax.numpy as jnp
from jax import lax
from jax.experimental import pallas as pl
from jax.experimental.pallas import tpu as pltpu
from jax.experimental.pallas import tpu_sc as plsc

N = 10000
E = 160000
IN = 128
OUT = 128

NC = 2
NS = 16
CHUNK = 64
E_PAD = 160768
R_ACC = 10240
EB = 4000


def _node_table_body(x_ref, w_ref, y_ref):
    y_ref[...] = jnp.dot(x_ref[...], w_ref[...],
                         preferred_element_type=jnp.float32)


def _edge_prep_body(e_ref, w_ref, ei_ref, a_ref, cnt_ref):
    a_ref[...] = jnp.dot(e_ref[...], w_ref[...],
                         preferred_element_type=jnp.float32)

    @pl.when(pl.program_id(0) == 0)
    def _init():
        cnt_ref[...] = jnp.zeros_like(cnt_ref)

    lanes = lax.broadcasted_iota(jnp.int32, (EB, 128), 1)
    for d in range(NC):
        tgt = ei_ref[:, 1 - d:2 - d]
        oh_col = (lax.bitwise_and(tgt, 127) == lanes).astype(jnp.float32)
        oh_row = (lax.shift_right_logical(tgt, 7) == lanes).astype(jnp.float32)
        cnt_ref[d] += lax.dot_general(oh_row, oh_col,
                                      (((0,), (0,)), ((), ())),
                                      preferred_element_type=jnp.float32)


def _sc_body(eidx_hbm, coef_hbm, y_hbm, zmsg_hbm, omsg_hbm,
             accm_sh, gi_v, si_v, cf_v, grows_v, mbuf_v, sem):
    c = lax.axis_index("c")
    s = lax.axis_index("s")
    tile_edges = E_PAD // NS
    n_chunks = tile_edges // CHUNK
    base = s * tile_edges
    rows_per_tile = R_ACC // NS
    r0 = s * rows_per_tile

    pltpu.sync_copy(zmsg_hbm.at[pl.ds(r0, rows_per_tile)],
                    accm_sh.at[pl.ds(r0, rows_per_tile)])
    plsc.subcore_barrier()

    is_fwd = c == 0

    def chunk_body(k, carry):
        eb = base + k * CHUNK
        pltpu.sync_copy(eidx_hbm.at[c, pl.ds(eb, CHUNK)], gi_v)
        pltpu.sync_copy(eidx_hbm.at[1 - c, pl.ds(eb, CHUNK)], si_v)
        pltpu.sync_copy(coef_hbm.at[pl.ds(eb, CHUNK)], cf_v)
        pltpu.async_copy(y_hbm.at[gi_v], grows_v, sem).wait()

        def edge_body(i, carry2):
            av = cf_v[i]
            a0 = jnp.where(is_fwd, av[0], av[2])
            a1 = jnp.where(is_fwd, av[1], av[3])
            for j in range(OUT // 16):
                g0 = grows_v[i, pl.ds(j * 16, 16)]
                g1 = grows_v[i, pl.ds(128 + j * 16, 16)]
                mbuf_v[i, pl.ds(j * 16, 16)] = a0 * g0 + a1 * g1
            return carry2

        lax.fori_loop(0, CHUNK, edge_body, 0)
        pltpu.sync_copy(mbuf_v, accm_sh.at[si_v], add=True)
        return carry

    lax.fori_loop(0, n_chunks, chunk_body, 0)
    plsc.subcore_barrier()

    pltpu.sync_copy(accm_sh.at[pl.ds(r0, rows_per_tile)],
                    omsg_hbm.at[c, pl.ds(r0, rows_per_tile)])


def _final_body(msg_ref, cnt_ref, x_ref, w_ref, b_ref, o_ref):
    c0 = cnt_ref[0]
    c1 = cnt_ref[1]
    h = (msg_ref[0] / jnp.maximum(c0, 1.0)
         + msg_ref[1] / jnp.maximum(c1, 1.0))
    h = h + lax.dot_general(x_ref[...], w_ref[...],
                            (((1,), (1,)), ((), ())),
                            preferred_element_type=jnp.float32)
    o_ref[...] = jnp.maximum(h + b_ref[...], 0.0)


def kernel(node_embs, edge_embs, edge_index, W_rb, W_rb_inv, W_basis,
           W_self_w, W_self_b):
    B = W_rb.shape[0]
    Wb = W_basis.reshape(B, IN, OUT)
    w_cat = jnp.concatenate([Wb[0], Wb[1]], axis=1)
    w_coef = jnp.zeros((IN, 16), jnp.float32)
    w_coef = w_coef.at[:, 0:2].set(W_rb.T).at[:, 2:4].set(W_rb_inv.T)

    x_pad = jnp.zeros((R_ACC, IN), jnp.float32).at[:N].set(node_embs)
    y_tab = pl.pallas_call(
        _node_table_body,
        grid=(10,),
        in_specs=[pl.BlockSpec((R_ACC // 10, IN), lambda i: (i, 0)),
                  pl.BlockSpec((IN, 2 * OUT), lambda i: (0, 0))],
        out_specs=pl.BlockSpec((R_ACC // 10, 2 * OUT), lambda i: (i, 0)),
        out_shape=jax.ShapeDtypeStruct((R_ACC, 2 * OUT), jnp.float32),
    )(x_pad, w_cat)

    eidx_t = edge_index.T
    coef, cnt = pl.pallas_call(
        _edge_prep_body,
        grid=(E // EB,),
        in_specs=[pl.BlockSpec((EB, IN), lambda i: (i, 0)),
                  pl.BlockSpec((IN, 16), lambda i: (0, 0)),
                  pl.BlockSpec((EB, 2), lambda i: (i, 0))],
        out_specs=[pl.BlockSpec((EB, 16), lambda i: (i, 0)),
                   pl.BlockSpec((NC, 128, 128), lambda i: (0, 0, 0))],
        out_shape=(jax.ShapeDtypeStruct((E, 16), jnp.float32),
                   jax.ShapeDtypeStruct((NC, 128, 128), jnp.float32)),
    )(edge_embs, w_coef, eidx_t)
    coef_pad = jnp.zeros((E_PAD, 16), jnp.float32).at[:E].set(coef)

    pad_idx = N + jnp.arange(E_PAD - E, dtype=jnp.int32) % (R_ACC - N)
    eidx_pad = jnp.concatenate(
        [edge_index, jnp.broadcast_to(pad_idx, (2, E_PAD - E))], axis=1)

    zeros_msg = jnp.zeros((R_ACC, OUT), jnp.float32)

    mesh = plsc.VectorSubcoreMesh(core_axis_name="c", subcore_axis_name="s",
                                  num_cores=NC, num_subcores=NS)
    msg = pl.kernel(
        _sc_body,
        out_type=jax.ShapeDtypeStruct((NC, R_ACC, OUT), jnp.float32),
        mesh=mesh,
        scratch_types=[
            pltpu.VMEM_SHARED((R_ACC, OUT), jnp.float32),
            pltpu.VMEM((CHUNK,), jnp.int32),
            pltpu.VMEM((CHUNK,), jnp.int32),
            pltpu.VMEM((CHUNK, 16), jnp.float32),
            pltpu.VMEM((CHUNK, 2 * OUT), jnp.float32),
            pltpu.VMEM((CHUNK, OUT), jnp.float32),
            pltpu.SemaphoreType.DMA,
        ],
    )(eidx_pad, coef_pad, y_tab, zeros_msg)

    cnt_flat = cnt.reshape(NC, 128 * 128, 1)

    out = pl.pallas_call(
        _final_body,
        grid=(10,),
        in_specs=[pl.BlockSpec((NC, N // 10, OUT), lambda i: (0, i, 0)),
                  pl.BlockSpec((NC, N // 10, 1), lambda i: (0, i, 0)),
                  pl.BlockSpec((N // 10, IN), lambda i: (i, 0)),
                  pl.BlockSpec((OUT, IN), lambda i: (0, 0)),
                  pl.BlockSpec((1, OUT), lambda i: (0, 0))],
        out_specs=pl.BlockSpec((N // 10, OUT), lambda i: (i, 0)),
        out_shape=jax.ShapeDtypeStruct((N, OUT), jnp.float32),
    )(msg, cnt_flat, node_embs, W_self_w, W_self_b.reshape(1, OUT))
    return out


# speedup vs baseline: 2.3358x; 1.0806x over previous
import jax
import jax.numpy as jnp
from jax import lax
from jax.experimental import pallas as pl
from jax.experimental.pallas import tpu as pltpu
from jax.experimental.pallas import tpu_sc as plsc

N = 10000
E = 160000
IN = 128
OUT = 128

NC = 2
NS = 16
CHUNK = 48
NCHT = 210
E_PAD = NS * CHUNK * NCHT
R_ACC = 10112
EB = 4000


def _node_table_body(x_ref, w_ref, y_ref):
    y_ref[...] = jnp.dot(x_ref[...], w_ref[...],
                         preferred_element_type=jnp.float32)


def _edge_prep_body(e_ref, w_ref, ei_ref, a_ref, cnt_ref):
    a_ref[...] = jnp.dot(e_ref[...], w_ref[...],
                         preferred_element_type=jnp.float32)

    @pl.when(pl.program_id(0) == 0)
    def _init():
        cnt_ref[...] = jnp.zeros_like(cnt_ref)

    lanes = lax.broadcasted_iota(jnp.int32, (EB, 128), 1)
    for d in range(NC):
        tgt = ei_ref[:, 1 - d:2 - d]
        oh_col = (lax.bitwise_and(tgt, 127) == lanes).astype(jnp.float32)
        oh_row = (lax.shift_right_logical(tgt, 7) == lanes).astype(jnp.float32)
        cnt_ref[d] += lax.dot_general(oh_row, oh_col,
                                      (((0,), (0,)), ((), ())),
                                      preferred_element_type=jnp.float32)


def _sc_body(src_hbm, dst_hbm, coef_hbm, y_hbm, zmsg_hbm, omsg_hbm,
             accm_sh, gi0_v, gi1_v, si0_v, si1_v, cf0_v, cf1_v,
             gr0_v, gr1_v, mbuf_v, gsem0, gsem1, ssem):
    c = lax.axis_index("c")
    s = lax.axis_index("s")
    tile_edges = E_PAD // NS
    base = s * tile_edges
    rows_per_tile = R_ACC // NS
    r0 = s * rows_per_tile

    is_fwd = c == 0
    pltpu.sync_copy(zmsg_hbm.at[pl.ds(r0, rows_per_tile)],
                    accm_sh.at[pl.ds(r0, rows_per_tile)])
    plsc.subcore_barrier()

    gis = (gi0_v, gi1_v)
    sis = (si0_v, si1_v)
    cfs = (cf0_v, cf1_v)
    grs = (gr0_v, gr1_v)
    gsems = (gsem0, gsem1)

    def fetch(b, k):
        eb = base + k * CHUNK

        @pl.when(is_fwd)
        def _f():
            pltpu.sync_copy(src_hbm.at[pl.ds(eb, CHUNK)], gis[b])
            pltpu.sync_copy(dst_hbm.at[pl.ds(eb, CHUNK)], sis[b])

        @pl.when(jnp.logical_not(is_fwd))
        def _r():
            pltpu.sync_copy(dst_hbm.at[pl.ds(eb, CHUNK)], gis[b])
            pltpu.sync_copy(src_hbm.at[pl.ds(eb, CHUNK)], sis[b])

        pltpu.sync_copy(coef_hbm.at[pl.ds(eb, CHUNK)], cfs[b])
        pltpu.async_copy(y_hbm.at[gis[b]], grs[b], gsems[b])

    def process(b):
        pltpu.make_async_copy(y_hbm.at[gis[b]], grs[b], gsems[b]).wait()
        cf_v = cfs[b]
        gv = grs[b]

        def edge_body(i, carry2):
            av = cf_v[i]
            a0 = jnp.where(is_fwd, av[0], av[2])
            a1 = jnp.where(is_fwd, av[1], av[3])
            for j in range(OUT // 16):
                g0 = gv[i, pl.ds(j * 16, 16)]
                g1 = gv[i, pl.ds(128 + j * 16, 16)]
                mbuf_v[i, pl.ds(j * 16, 16)] = a0 * g0 + a1 * g1
            return carry2

        lax.fori_loop(0, CHUNK, edge_body, 0)
        pltpu.async_copy(mbuf_v, accm_sh.at[sis[b]], ssem, add=True).wait()

    fetch(0, 0)

    def pair_body(p, carry):
        k = 2 * p
        fetch(1, k + 1)
        process(0)

        @pl.when(k + 2 < NCHT)
        def _f0():
            fetch(0, k + 2)

        process(1)
        return carry

    lax.fori_loop(0, NCHT // 2, pair_body, 0)
    plsc.subcore_barrier()

    pltpu.sync_copy(accm_sh.at[pl.ds(r0, rows_per_tile)],
                    omsg_hbm.at[c, pl.ds(r0, rows_per_tile)])


def _final_body(msg_ref, cnt_ref, x_ref, w_ref, b_ref, o_ref):
    c0 = cnt_ref[0]
    c1 = cnt_ref[1]
    h = (msg_ref[0] / jnp.maximum(c0, 1.0)
         + msg_ref[1] / jnp.maximum(c1, 1.0))
    h = h + lax.dot_general(x_ref[...], w_ref[...],
                            (((1,), (1,)), ((), ())),
                            preferred_element_type=jnp.float32)
    o_ref[...] = jnp.maximum(h + b_ref[...], 0.0)


def kernel(node_embs, edge_embs, edge_index, W_rb, W_rb_inv, W_basis,
           W_self_w, W_self_b):
    B = W_rb.shape[0]
    Wb = W_basis.reshape(B, IN, OUT)
    w_cat = jnp.concatenate([Wb[0], Wb[1]], axis=1)
    w_coef = jnp.zeros((IN, 16), jnp.float32)
    w_coef = w_coef.at[:, 0:2].set(W_rb.T).at[:, 2:4].set(W_rb_inv.T)

    x_pad = jnp.zeros((R_ACC, IN), jnp.float32).at[:N].set(node_embs)
    y_tab = pl.pallas_call(
        _node_table_body,
        grid=(8,),
        in_specs=[pl.BlockSpec((R_ACC // 8, IN), lambda i: (i, 0)),
                  pl.BlockSpec((IN, 2 * OUT), lambda i: (0, 0))],
        out_specs=pl.BlockSpec((R_ACC // 8, 2 * OUT), lambda i: (i, 0)),
        out_shape=jax.ShapeDtypeStruct((R_ACC, 2 * OUT), jnp.float32),
    )(x_pad, w_cat)

    eidx_t = edge_index.T
    coef, cnt = pl.pallas_call(
        _edge_prep_body,
        grid=(E // EB,),
        in_specs=[pl.BlockSpec((EB, IN), lambda i: (i, 0)),
                  pl.BlockSpec((IN, 16), lambda i: (0, 0)),
                  pl.BlockSpec((EB, 2), lambda i: (i, 0))],
        out_specs=[pl.BlockSpec((EB, 16), lambda i: (i, 0)),
                   pl.BlockSpec((NC, 128, 128), lambda i: (0, 0, 0))],
        out_shape=(jax.ShapeDtypeStruct((E, 16), jnp.float32),
                   jax.ShapeDtypeStruct((NC, 128, 128), jnp.float32)),
    )(edge_embs, w_coef, eidx_t)
    coef_pad = jnp.zeros((E_PAD, 16), jnp.float32).at[:E].set(coef)

    pad_idx = N + jnp.arange(E_PAD - E, dtype=jnp.int32) % (R_ACC - N)
    src_pad = jnp.concatenate([edge_index[0], pad_idx])
    dst_pad = jnp.concatenate([edge_index[1], pad_idx])

    zeros_msg = jnp.zeros((R_ACC, OUT), jnp.float32)

    mesh = plsc.VectorSubcoreMesh(core_axis_name="c", subcore_axis_name="s",
                                  num_cores=NC, num_subcores=NS)
    msg = pl.kernel(
        _sc_body,
        out_type=jax.ShapeDtypeStruct((NC, R_ACC, OUT), jnp.float32),
        mesh=mesh,
        scratch_types=[
            pltpu.VMEM_SHARED((R_ACC, OUT), jnp.float32),
            pltpu.VMEM((CHUNK,), jnp.int32),
            pltpu.VMEM((CHUNK,), jnp.int32),
            pltpu.VMEM((CHUNK,), jnp.int32),
            pltpu.VMEM((CHUNK,), jnp.int32),
            pltpu.VMEM((CHUNK, 16), jnp.float32),
            pltpu.VMEM((CHUNK, 16), jnp.float32),
            pltpu.VMEM((CHUNK, 2 * OUT), jnp.float32),
            pltpu.VMEM((CHUNK, 2 * OUT), jnp.float32),
            pltpu.VMEM((CHUNK, OUT), jnp.float32),
            pltpu.SemaphoreType.DMA,
            pltpu.SemaphoreType.DMA,
            pltpu.SemaphoreType.DMA,
        ],
    )(src_pad, dst_pad, coef_pad, y_tab, zeros_msg)

    cnt_flat = cnt.reshape(NC, 128 * 128, 1)

    out = pl.pallas_call(
        _final_body,
        grid=(10,),
        in_specs=[pl.BlockSpec((NC, N // 10, OUT), lambda i: (0, i, 0)),
                  pl.BlockSpec((NC, N // 10, 1), lambda i: (0, i, 0)),
                  pl.BlockSpec((N // 10, IN), lambda i: (i, 0)),
                  pl.BlockSpec((OUT, IN), lambda i: (0, 0)),
                  pl.BlockSpec((1, OUT), lambda i: (0, 0))],
        out_specs=pl.BlockSpec((N // 10, OUT), lambda i: (i, 0)),
        out_shape=jax.ShapeDtypeStruct((N, OUT), jnp.float32),
    )(msg, cnt_flat, node_embs, W_self_w, W_self_b.reshape(1, OUT))
    return out


# speedup vs baseline: 2.3429x; 1.0031x over previous
import jax
import jax.numpy as jnp
from jax import lax
from jax.experimental import pallas as pl
from jax.experimental.pallas import tpu as pltpu
from jax.experimental.pallas import tpu_sc as plsc

N = 10000
E = 160000
IN = 128
OUT = 128

NC = 2
NS = 16
CHUNK = 48
NCHT = 210
E_PAD = NS * CHUNK * NCHT
R_ACC = 10112
EB = 4000


def _node_table_body(x_ref, w_ref, y_ref):
    y_ref[...] = jnp.dot(x_ref[...], w_ref[...],
                         preferred_element_type=jnp.float32)


def _edge_prep_body(e_ref, w_ref, ei_ref, a_ref, cnt_ref):
    a_ref[...] = jnp.dot(e_ref[...], w_ref[...],
                         preferred_element_type=jnp.float32)

    @pl.when(pl.program_id(0) == 0)
    def _init():
        cnt_ref[...] = jnp.zeros_like(cnt_ref)

    lanes = lax.broadcasted_iota(jnp.int32, (EB, 128), 1)
    for d in range(NC):
        tgt = ei_ref[:, 1 - d:2 - d]
        oh_col = (lax.bitwise_and(tgt, 127) == lanes).astype(jnp.float32)
        oh_row = (lax.shift_right_logical(tgt, 7) == lanes).astype(jnp.float32)
        cnt_ref[d] += lax.dot_general(oh_row, oh_col,
                                      (((0,), (0,)), ((), ())),
                                      preferred_element_type=jnp.float32)


def _sc_body(src_hbm, dst_hbm, coef_hbm, y_hbm, zmsg_hbm, omsg_hbm,
             accm_sh, gi0_v, gi1_v, si0_v, si1_v, cf0_v, cf1_v,
             gr0_v, gr1_v, mbuf_v, gsem0, gsem1, ssem):
    c = lax.axis_index("c")
    s = lax.axis_index("s")
    tile_edges = E_PAD // NS
    base = s * tile_edges
    rows_per_tile = R_ACC // NS
    r0 = s * rows_per_tile

    is_fwd = c == 0
    pltpu.sync_copy(zmsg_hbm.at[pl.ds(r0, rows_per_tile)],
                    accm_sh.at[pl.ds(r0, rows_per_tile)])
    plsc.subcore_barrier()

    gis = (gi0_v, gi1_v)
    sis = (si0_v, si1_v)
    cfs = (cf0_v, cf1_v)
    grs = (gr0_v, gr1_v)
    gsems = (gsem0, gsem1)

    def fetch(b, k):
        eb = base + k * CHUNK

        @pl.when(is_fwd)
        def _f():
            pltpu.sync_copy(src_hbm.at[pl.ds(eb, CHUNK)], gis[b])
            pltpu.sync_copy(dst_hbm.at[pl.ds(eb, CHUNK)], sis[b])

        @pl.when(jnp.logical_not(is_fwd))
        def _r():
            pltpu.sync_copy(dst_hbm.at[pl.ds(eb, CHUNK)], gis[b])
            pltpu.sync_copy(src_hbm.at[pl.ds(eb, CHUNK)], sis[b])

        pltpu.sync_copy(coef_hbm.at[pl.ds(eb, CHUNK)], cfs[b])
        pltpu.async_copy(y_hbm.at[gis[b]], grs[b], gsems[b])

    def process(b):
        pltpu.make_async_copy(y_hbm.at[gis[b]], grs[b], gsems[b]).wait()
        cf_v = cfs[b]
        gv = grs[b]

        def edge_body(i, carry2):
            av = cf_v[i]
            a0 = jnp.where(is_fwd, av[0], av[2])
            a1 = jnp.where(is_fwd, av[1], av[3])
            for j in range(OUT // 16):
                g0 = gv[i, pl.ds(j * 16, 16)]
                g1 = gv[i, pl.ds(128 + j * 16, 16)]
                mbuf_v[i, pl.ds(j * 16, 16)] = a0 * g0 + a1 * g1
            return carry2

        lax.fori_loop(0, CHUNK, edge_body, 0, unroll=4)
        pltpu.async_copy(mbuf_v, accm_sh.at[sis[b]], ssem, add=True).wait()

    fetch(0, 0)

    def pair_body(p, carry):
        k = 2 * p
        fetch(1, k + 1)
        process(0)

        @pl.when(k + 2 < NCHT)
        def _f0():
            fetch(0, k + 2)

        process(1)
        return carry

    lax.fori_loop(0, NCHT // 2, pair_body, 0)
    plsc.subcore_barrier()

    pltpu.sync_copy(accm_sh.at[pl.ds(r0, rows_per_tile)],
                    omsg_hbm.at[c, pl.ds(r0, rows_per_tile)])


def _final_body(msg_ref, cnt_ref, x_ref, w_ref, b_ref, o_ref):
    c0 = cnt_ref[0]
    c1 = cnt_ref[1]
    h = (msg_ref[0] / jnp.maximum(c0, 1.0)
         + msg_ref[1] / jnp.maximum(c1, 1.0))
    h = h + lax.dot_general(x_ref[...], w_ref[...],
                            (((1,), (1,)), ((), ())),
                            preferred_element_type=jnp.float32)
    o_ref[...] = jnp.maximum(h + b_ref[...], 0.0)


def kernel(node_embs, edge_embs, edge_index, W_rb, W_rb_inv, W_basis,
           W_self_w, W_self_b):
    B = W_rb.shape[0]
    Wb = W_basis.reshape(B, IN, OUT)
    w_cat = jnp.concatenate([Wb[0], Wb[1]], axis=1)
    w_coef = jnp.zeros((IN, 16), jnp.float32)
    w_coef = w_coef.at[:, 0:2].set(W_rb.T).at[:, 2:4].set(W_rb_inv.T)

    x_pad = jnp.zeros((R_ACC, IN), jnp.float32).at[:N].set(node_embs)
    y_tab = pl.pallas_call(
        _node_table_body,
        grid=(8,),
        in_specs=[pl.BlockSpec((R_ACC // 8, IN), lambda i: (i, 0)),
                  pl.BlockSpec((IN, 2 * OUT), lambda i: (0, 0))],
        out_specs=pl.BlockSpec((R_ACC // 8, 2 * OUT), lambda i: (i, 0)),
        out_shape=jax.ShapeDtypeStruct((R_ACC, 2 * OUT), jnp.float32),
    )(x_pad, w_cat)

    eidx_t = edge_index.T
    coef, cnt = pl.pallas_call(
        _edge_prep_body,
        grid=(E // EB,),
        in_specs=[pl.BlockSpec((EB, IN), lambda i: (i, 0)),
                  pl.BlockSpec((IN, 16), lambda i: (0, 0)),
                  pl.BlockSpec((EB, 2), lambda i: (i, 0))],
        out_specs=[pl.BlockSpec((EB, 16), lambda i: (i, 0)),
                   pl.BlockSpec((NC, 128, 128), lambda i: (0, 0, 0))],
        out_shape=(jax.ShapeDtypeStruct((E, 16), jnp.float32),
                   jax.ShapeDtypeStruct((NC, 128, 128), jnp.float32)),
    )(edge_embs, w_coef, eidx_t)
    coef_pad = jnp.zeros((E_PAD, 16), jnp.float32).at[:E].set(coef)

    pad_idx = N + jnp.arange(E_PAD - E, dtype=jnp.int32) % (R_ACC - N)
    src_pad = jnp.concatenate([edge_index[0], pad_idx])
    dst_pad = jnp.concatenate([edge_index[1], pad_idx])

    zeros_msg = jnp.zeros((R_ACC, OUT), jnp.float32)

    mesh = plsc.VectorSubcoreMesh(core_axis_name="c", subcore_axis_name="s",
                                  num_cores=NC, num_subcores=NS)
    msg = pl.kernel(
        _sc_body,
        out_type=jax.ShapeDtypeStruct((NC, R_ACC, OUT), jnp.float32),
        mesh=mesh,
        scratch_types=[
            pltpu.VMEM_SHARED((R_ACC, OUT), jnp.float32),
            pltpu.VMEM((CHUNK,), jnp.int32),
            pltpu.VMEM((CHUNK,), jnp.int32),
            pltpu.VMEM((CHUNK,), jnp.int32),
            pltpu.VMEM((CHUNK,), jnp.int32),
            pltpu.VMEM((CHUNK, 16), jnp.float32),
            pltpu.VMEM((CHUNK, 16), jnp.float32),
            pltpu.VMEM((CHUNK, 2 * OUT), jnp.float32),
            pltpu.VMEM((CHUNK, 2 * OUT), jnp.float32),
            pltpu.VMEM((CHUNK, OUT), jnp.float32),
            pltpu.SemaphoreType.DMA,
            pltpu.SemaphoreType.DMA,
            pltpu.SemaphoreType.DMA,
        ],
    )(src_pad, dst_pad, coef_pad, y_tab, zeros_msg)

    cnt_flat = cnt.reshape(NC, 128 * 128, 1)

    out = pl.pallas_call(
        _final_body,
        grid=(10,),
        in_specs=[pl.BlockSpec((NC, N // 10, OUT), lambda i: (0, i, 0)),
                  pl.BlockSpec((NC, N // 10, 1), lambda i: (0, i, 0)),
                  pl.BlockSpec((N // 10, IN), lambda i: (i, 0)),
                  pl.BlockSpec((OUT, IN), lambda i: (0, 0)),
                  pl.BlockSpec((1, OUT), lambda i: (0, 0))],
        out_specs=pl.BlockSpec((N // 10, OUT), lambda i: (i, 0)),
        out_shape=jax.ShapeDtypeStruct((N, OUT), jnp.float32),
    )(msg, cnt_flat, node_embs, W_self_w, W_self_b.reshape(1, OUT))
    return out


# speedup vs baseline: 2.9106x; 1.2423x over previous
import jax
import jax.numpy as jnp
from jax import lax
from jax.experimental import pallas as pl
from jax.experimental.pallas import tpu as pltpu
from jax.experimental.pallas import tpu_sc as plsc

N = 10000
E = 160000
IN = 128
OUT = 128

NC = 2
NS = 16
CHUNK = 48
NCHT = 210
E_PAD = NS * CHUNK * NCHT
R_ACC = 10112
EB = 4000


def _node_table_body(x_ref, w_ref, y_ref):
    y_ref[...] = jnp.dot(x_ref[...], w_ref[...],
                         preferred_element_type=jnp.float32)


def _edge_prep_body(e_ref, w_ref, ei_ref, a_ref, cnt_ref):
    a_ref[...] = jnp.dot(e_ref[...], w_ref[...],
                         preferred_element_type=jnp.float32)

    @pl.when(pl.program_id(0) == 0)
    def _init():
        cnt_ref[...] = jnp.zeros_like(cnt_ref)

    lanes = lax.broadcasted_iota(jnp.int32, (EB, 128), 1)
    for d in range(NC):
        tgt = ei_ref[:, 1 - d:2 - d]
        oh_col = (lax.bitwise_and(tgt, 127) == lanes).astype(jnp.float32)
        oh_row = (lax.shift_right_logical(tgt, 7) == lanes).astype(jnp.float32)
        cnt_ref[d] += lax.dot_general(oh_row, oh_col,
                                      (((0,), (0,)), ((), ())),
                                      preferred_element_type=jnp.float32)


def _sc_body(src_hbm, dst_hbm, coef_hbm, y_hbm, zmsg_hbm, omsg_hbm,
             accm_sh, gi0_v, gi1_v, si0_v, si1_v, cf0_v, cf1_v,
             gr0_v, gr1_v, mbuf_v, gsem0, gsem1, rsem0, rsem1, ssem):
    c = lax.axis_index("c")
    s = lax.axis_index("s")
    tile_edges = E_PAD // NS
    base = s * tile_edges
    rows_per_tile = R_ACC // NS
    r0 = s * rows_per_tile

    is_fwd = c == 0
    pltpu.sync_copy(zmsg_hbm.at[pl.ds(r0, rows_per_tile)],
                    accm_sh.at[pl.ds(r0, rows_per_tile)])
    plsc.subcore_barrier()

    gis = (gi0_v, gi1_v)
    sis = (si0_v, si1_v)
    cfs = (cf0_v, cf1_v)
    grs = (gr0_v, gr1_v)
    gsems = (gsem0, gsem1)
    rsems = (rsem0, rsem1)

    def fetch(b, k):
        eb = base + k * CHUNK

        @pl.when(is_fwd)
        def _f():
            pltpu.sync_copy(src_hbm.at[pl.ds(eb, CHUNK)], gis[b])
            pltpu.async_copy(dst_hbm.at[pl.ds(eb, CHUNK)], sis[b], rsems[b])

        @pl.when(jnp.logical_not(is_fwd))
        def _r():
            pltpu.sync_copy(dst_hbm.at[pl.ds(eb, CHUNK)], gis[b])
            pltpu.async_copy(src_hbm.at[pl.ds(eb, CHUNK)], sis[b], rsems[b])

        pltpu.async_copy(coef_hbm.at[pl.ds(eb, CHUNK)], cfs[b], rsems[b])
        pltpu.async_copy(y_hbm.at[gis[b]], grs[b], gsems[b])

    def process(b):
        pltpu.make_async_copy(y_hbm.at[gis[b]], grs[b], gsems[b]).wait()
        pltpu.make_async_copy(coef_hbm.at[pl.ds(0, CHUNK)], cfs[b],
                              rsems[b]).wait()
        pltpu.make_async_copy(src_hbm.at[pl.ds(0, CHUNK)], sis[b],
                              rsems[b]).wait()
        cf_v = cfs[b]
        gv = grs[b]

        def edge_body(i, carry2):
            av = cf_v[i]
            a0 = jnp.where(is_fwd, av[0], av[2])
            a1 = jnp.where(is_fwd, av[1], av[3])
            for j in range(OUT // 16):
                g0 = gv[i, pl.ds(j * 16, 16)]
                g1 = gv[i, pl.ds(128 + j * 16, 16)]
                mbuf_v[i, pl.ds(j * 16, 16)] = a0 * g0 + a1 * g1
            return carry2

        lax.fori_loop(0, CHUNK, edge_body, 0, unroll=4)
        pltpu.async_copy(mbuf_v, accm_sh.at[sis[b]], ssem, add=True).wait()

    fetch(0, 0)

    def pair_body(p, carry):
        k = 2 * p
        fetch(1, k + 1)
        process(0)

        @pl.when(k + 2 < NCHT)
        def _f0():
            fetch(0, k + 2)

        process(1)
        return carry

    lax.fori_loop(0, NCHT // 2, pair_body, 0)
    plsc.subcore_barrier()

    pltpu.sync_copy(accm_sh.at[pl.ds(r0, rows_per_tile)],
                    omsg_hbm.at[c, pl.ds(r0, rows_per_tile)])


def _final_body(msg_ref, cnt_ref, x_ref, w_ref, b_ref, o_ref):
    c0 = cnt_ref[0]
    c1 = cnt_ref[1]
    h = (msg_ref[0] / jnp.maximum(c0, 1.0)
         + msg_ref[1] / jnp.maximum(c1, 1.0))
    h = h + lax.dot_general(x_ref[...], w_ref[...],
                            (((1,), (1,)), ((), ())),
                            preferred_element_type=jnp.float32)
    o_ref[...] = jnp.maximum(h + b_ref[...], 0.0)


def kernel(node_embs, edge_embs, edge_index, W_rb, W_rb_inv, W_basis,
           W_self_w, W_self_b):
    B = W_rb.shape[0]
    Wb = W_basis.reshape(B, IN, OUT)
    w_cat = jnp.concatenate([Wb[0], Wb[1]], axis=1)
    w_coef = jnp.zeros((IN, 16), jnp.float32)
    w_coef = w_coef.at[:, 0:2].set(W_rb.T).at[:, 2:4].set(W_rb_inv.T)

    x_pad = jnp.zeros((R_ACC, IN), jnp.float32).at[:N].set(node_embs)
    y_tab = pl.pallas_call(
        _node_table_body,
        grid=(8,),
        in_specs=[pl.BlockSpec((R_ACC // 8, IN), lambda i: (i, 0)),
                  pl.BlockSpec((IN, 2 * OUT), lambda i: (0, 0))],
        out_specs=pl.BlockSpec((R_ACC // 8, 2 * OUT), lambda i: (i, 0)),
        out_shape=jax.ShapeDtypeStruct((R_ACC, 2 * OUT), jnp.float32),
    )(x_pad, w_cat)

    eidx_t = edge_index.T
    coef, cnt = pl.pallas_call(
        _edge_prep_body,
        grid=(E // EB,),
        in_specs=[pl.BlockSpec((EB, IN), lambda i: (i, 0)),
                  pl.BlockSpec((IN, 16), lambda i: (0, 0)),
                  pl.BlockSpec((EB, 2), lambda i: (i, 0))],
        out_specs=[pl.BlockSpec((EB, 16), lambda i: (i, 0)),
                   pl.BlockSpec((NC, 128, 128), lambda i: (0, 0, 0))],
        out_shape=(jax.ShapeDtypeStruct((E, 16), jnp.float32),
                   jax.ShapeDtypeStruct((NC, 128, 128), jnp.float32)),
    )(edge_embs, w_coef, eidx_t)
    coef_pad = jnp.zeros((E_PAD, 16), jnp.float32).at[:E].set(coef)

    pad_idx = N + jnp.arange(E_PAD - E, dtype=jnp.int32) % (R_ACC - N)
    src_pad = jnp.concatenate([edge_index[0], pad_idx])
    dst_pad = jnp.concatenate([edge_index[1], pad_idx])

    zeros_msg = jnp.zeros((R_ACC, OUT), jnp.float32)

    mesh = plsc.VectorSubcoreMesh(core_axis_name="c", subcore_axis_name="s",
                                  num_cores=NC, num_subcores=NS)
    msg = pl.kernel(
        _sc_body,
        out_type=jax.ShapeDtypeStruct((NC, R_ACC, OUT), jnp.float32),
        mesh=mesh,
        scratch_types=[
            pltpu.VMEM_SHARED((R_ACC, OUT), jnp.float32),
            pltpu.VMEM((CHUNK,), jnp.int32),
            pltpu.VMEM((CHUNK,), jnp.int32),
            pltpu.VMEM((CHUNK,), jnp.int32),
            pltpu.VMEM((CHUNK,), jnp.int32),
            pltpu.VMEM((CHUNK, 16), jnp.float32),
            pltpu.VMEM((CHUNK, 16), jnp.float32),
            pltpu.VMEM((CHUNK, 2 * OUT), jnp.float32),
            pltpu.VMEM((CHUNK, 2 * OUT), jnp.float32),
            pltpu.VMEM((CHUNK, OUT), jnp.float32),
            pltpu.SemaphoreType.DMA,
            pltpu.SemaphoreType.DMA,
            pltpu.SemaphoreType.DMA,
            pltpu.SemaphoreType.DMA,
            pltpu.SemaphoreType.DMA,
        ],
    )(src_pad, dst_pad, coef_pad, y_tab, zeros_msg)

    cnt_flat = cnt.reshape(NC, 128 * 128, 1)

    out = pl.pallas_call(
        _final_body,
        grid=(10,),
        in_specs=[pl.BlockSpec((NC, N // 10, OUT), lambda i: (0, i, 0)),
                  pl.BlockSpec((NC, N // 10, 1), lambda i: (0, i, 0)),
                  pl.BlockSpec((N // 10, IN), lambda i: (i, 0)),
                  pl.BlockSpec((OUT, IN), lambda i: (0, 0)),
                  pl.BlockSpec((1, OUT), lambda i: (0, 0))],
        out_specs=pl.BlockSpec((N // 10, OUT), lambda i: (i, 0)),
        out_shape=jax.ShapeDtypeStruct((N, OUT), jnp.float32),
    )(msg, cnt_flat, node_embs, W_self_w, W_self_b.reshape(1, OUT))
    return out
